# Initial kernel scaffold; baseline (speedup 1.0000x reference)
import functools

import jax
import jax.numpy as jnp
from jax.experimental import pallas as pl
from jax.experimental.pallas import tpu as pltpu


def _conv_pool_body(x_ref, w_ref, b_ref, o_ref, acc_e, acc_o, *,
                    band, w2, k5, cout):
    for kh in range(5):
        wk = w_ref[kh]
        for ph, acc in ((0, acc_e), (1, acc_o)):
            lhs = x_ref[pl.ds(kh, band), ph, :, :]
            c = jnp.dot(lhs.reshape(band * w2, k5), wk,
                        preferred_element_type=jnp.float32)
            c = c.reshape(band, w2, cout)
            if kh == 0:
                acc[...] = c
            else:
                acc[...] += c

    bias = b_ref[...]
    ae = jnp.maximum(acc_e[...] + bias, 0.0)
    ao = jnp.maximum(acc_o[...] + bias, 0.0)
    pw = jnp.maximum(ae, ao)
    o_ref[...] = pw.reshape(band // 2, 2, w2, cout).max(axis=1)


def _kw_preshift(x_nhwc):
    n, h, w, c = x_nhwc.shape
    w2 = w // 2
    xp = jnp.pad(x_nhwc, ((0, 0), (2, 2), (2, 2), (0, 0)))
    phs = (xp[:, :, 0::2, :], xp[:, :, 1::2, :])
    cols = []
    for ph in range(2):
        chunks = [phs[(ph + kw) % 2][:, :, (ph + kw) // 2:(ph + kw) // 2 + w2, :]
                  for kw in range(5)]
        cols.append(jnp.concatenate(chunks, axis=-1))
    return jnp.stack(cols, axis=2)


def _conv_layer(x_nhwc, w_taps, b_row, *, band):
    n, h, w, cin = x_nhwc.shape
    cout = w_taps.shape[-1]
    h2, w2 = h // 2, w // 2
    nb = h // band
    k5 = 5 * cin

    x5 = _kw_preshift(x_nhwc)
    w5 = w_taps.reshape(5, 5, cin, cout).reshape(5, k5, cout)
    body = functools.partial(_conv_pool_body, band=band, w2=w2, k5=k5,
                             cout=cout)
    params = pltpu.CompilerParams(
        dimension_semantics=("parallel",) if nb == 1 else ("parallel", "arbitrary"),
        vmem_limit_bytes=48 * 1024 * 1024)
    scratch = [pltpu.VMEM((band, w2, cout), jnp.float32),
               pltpu.VMEM((band, w2, cout), jnp.float32)]
    out_shape = jax.ShapeDtypeStruct((n, h2, w2, cout), jnp.float32)

    if nb == 1:
        return pl.pallas_call(
            body,
            out_shape=out_shape,
            grid=(n,),
            in_specs=[
                pl.BlockSpec((None, band + 4, 2, w2, k5), lambda ni: (ni, 0, 0, 0, 0)),
                pl.BlockSpec((5, k5, cout), lambda ni: (0, 0, 0)),
                pl.BlockSpec((1, cout), lambda ni: (0, 0)),
            ],
            out_specs=pl.BlockSpec((None, band // 2, w2, cout),
                                   lambda ni: (ni, 0, 0, 0)),
            scratch_shapes=scratch,
            compiler_params=params,
        )(x5, w5, b_row)

    xw = jnp.stack([x5[:, i * band:i * band + band + 4] for i in range(nb)],
                   axis=1)
    return pl.pallas_call(
        body,
        out_shape=out_shape,
        grid=(n, nb),
        in_specs=[
            pl.BlockSpec((None, None, band + 4, 2, w2, k5),
                         lambda ni, bi: (ni, bi, 0, 0, 0, 0)),
            pl.BlockSpec((5, k5, cout), lambda ni, bi: (0, 0, 0)),
            pl.BlockSpec((1, cout), lambda ni, bi: (0, 0)),
        ],
        out_specs=pl.BlockSpec((None, band // 2, w2, cout),
                               lambda ni, bi: (ni, bi, 0, 0)),
        scratch_shapes=scratch,
        compiler_params=params,
    )(xw, w5, b_row)


def _mlp_body(x_ref, w1_ref, b1_ref, w2_ref, b2_ref, o_ref, acc_ref):
    k = pl.program_id(0)

    @pl.when(k == 0)
    def _():
        acc_ref[...] = jnp.zeros_like(acc_ref)

    acc_ref[...] += jnp.dot(x_ref[...], w1_ref[...],
                            preferred_element_type=jnp.float32)

    @pl.when(k == pl.num_programs(0) - 1)
    def _():
        h = jnp.maximum(acc_ref[...] + b1_ref[...], 0.0)
        o_ref[...] = jnp.dot(h, w2_ref[...],
                             preferred_element_type=jnp.float32) + b2_ref[...]


def _mlp_head(feats, w1, b1, w2, b2, *, tk):
    n, kdim = feats.shape
    h1 = w1.shape[1]
    o = w2.shape[1]
    return pl.pallas_call(
        _mlp_body,
        out_shape=jax.ShapeDtypeStruct((n, o), jnp.float32),
        grid=(kdim // tk,),
        in_specs=[
            pl.BlockSpec((n, tk), lambda k: (0, k)),
            pl.BlockSpec((tk, h1), lambda k: (k, 0)),
            pl.BlockSpec((1, h1), lambda k: (0, 0)),
            pl.BlockSpec((h1, o), lambda k: (0, 0)),
            pl.BlockSpec((1, o), lambda k: (0, 0)),
        ],
        out_specs=pl.BlockSpec((n, o), lambda k: (0, 0)),
        scratch_shapes=[pltpu.VMEM((n, h1), jnp.float32)],
        compiler_params=pltpu.CompilerParams(
            dimension_semantics=("arbitrary",),
            vmem_limit_bytes=48 * 1024 * 1024),
    )(feats, w1, b1, w2, b2)


def kernel(x_nchw, c1w, c1b, c2w, c2b, c3w, c3b, f1w, f1b, f2w, f2b):
    x = jnp.transpose(x_nchw, (0, 2, 3, 1))
    x = _conv_layer(x, c1w, c1b, band=56)
    x = _conv_layer(x, c2w, c2b, band=112)
    x = _conv_layer(x, c3w, c3b, band=56)
    feats = x.reshape(x.shape[0], -1)
    return _mlp_head(feats, f1w, f1b, f2w, f2b, tk=23520)


# speedup vs baseline: 1.2171x; 1.2171x over previous
import functools

import jax
import jax.numpy as jnp
from jax.experimental import pallas as pl
from jax.experimental.pallas import tpu as pltpu


def _conv_pool_body(x_ref, w_ref, b_ref, o_ref, acc_e, acc_o, *,
                    band, w2, k5, cout):
    for kh in range(5):
        wk = w_ref[kh]
        for ph, acc in ((0, acc_e), (1, acc_o)):
            lhs = x_ref[pl.ds(kh, band), ph, :, :]
            c = jnp.dot(lhs.reshape(band * w2, k5), wk,
                        preferred_element_type=jnp.float32)
            c = c.reshape(band, w2, cout)
            if kh == 0:
                acc[...] = c
            else:
                acc[...] += c

    bias = b_ref[...]
    ae = jnp.maximum(acc_e[...] + bias, 0.0)
    ao = jnp.maximum(acc_o[...] + bias, 0.0)
    pw = jnp.maximum(ae, ao)
    o_ref[...] = pw.reshape(band // 2, 2, w2, cout).max(axis=1)


def _kw_preshift(x_nhwc):
    n, h, w, c = x_nhwc.shape
    w2 = w // 2
    xp = jnp.pad(x_nhwc, ((0, 0), (2, 2), (2, 2), (0, 0)))
    phs = (xp[:, :, 0::2, :], xp[:, :, 1::2, :])
    cols = []
    for ph in range(2):
        chunks = [phs[(ph + kw) % 2][:, :, (ph + kw) // 2:(ph + kw) // 2 + w2, :]
                  for kw in range(5)]
        cols.append(jnp.concatenate(chunks, axis=-1))
    return jnp.stack(cols, axis=2)


def _conv_layer(x_nhwc, w_taps, b_row, *, band):
    n, h, w, cin = x_nhwc.shape
    cout = w_taps.shape[-1]
    h2, w2 = h // 2, w // 2
    nb = h // band
    k5 = 5 * cin

    x5 = _kw_preshift(x_nhwc)
    w5 = w_taps.reshape(5, 5, cin, cout).reshape(5, k5, cout)
    body = functools.partial(_conv_pool_body, band=band, w2=w2, k5=k5,
                             cout=cout)
    params = pltpu.CompilerParams(
        dimension_semantics=("parallel",) if nb == 1 else ("parallel", "arbitrary"),
        vmem_limit_bytes=48 * 1024 * 1024)
    scratch = [pltpu.VMEM((band, w2, cout), jnp.float32),
               pltpu.VMEM((band, w2, cout), jnp.float32)]
    out_shape = jax.ShapeDtypeStruct((n, h2, w2, cout), jnp.float32)

    if nb == 1:
        return pl.pallas_call(
            body,
            out_shape=out_shape,
            grid=(n,),
            in_specs=[
                pl.BlockSpec((None, band + 4, 2, w2, k5), lambda ni: (ni, 0, 0, 0, 0)),
                pl.BlockSpec((5, k5, cout), lambda ni: (0, 0, 0)),
                pl.BlockSpec((1, cout), lambda ni: (0, 0)),
            ],
            out_specs=pl.BlockSpec((None, band // 2, w2, cout),
                                   lambda ni: (ni, 0, 0, 0)),
            scratch_shapes=scratch,
            compiler_params=params,
        )(x5, w5, b_row)

    xw = jnp.stack([x5[:, i * band:i * band + band + 4] for i in range(nb)],
                   axis=1)
    return pl.pallas_call(
        body,
        out_shape=out_shape,
        grid=(n, nb),
        in_specs=[
            pl.BlockSpec((None, None, band + 4, 2, w2, k5),
                         lambda ni, bi: (ni, bi, 0, 0, 0, 0)),
            pl.BlockSpec((5, k5, cout), lambda ni, bi: (0, 0, 0)),
            pl.BlockSpec((1, cout), lambda ni, bi: (0, 0)),
        ],
        out_specs=pl.BlockSpec((None, band // 2, w2, cout),
                               lambda ni, bi: (ni, bi, 0, 0)),
        scratch_shapes=scratch,
        compiler_params=params,
    )(xw, w5, b_row)


def _mlp_body(x_ref, w1_ref, b1_ref, w2_ref, b2_ref, o_ref, acc_ref):
    k = pl.program_id(0)

    @pl.when(k == 0)
    def _():
        acc_ref[...] = jnp.zeros_like(acc_ref)

    acc_ref[...] += jnp.dot(x_ref[...], w1_ref[...],
                            preferred_element_type=jnp.float32)

    @pl.when(k == pl.num_programs(0) - 1)
    def _():
        h = jnp.maximum(acc_ref[...] + b1_ref[...], 0.0)
        o_ref[...] = jnp.dot(h, w2_ref[...],
                             preferred_element_type=jnp.float32) + b2_ref[...]


def _mlp_head(feats, w1, b1, w2, b2, *, tk):
    n, kdim = feats.shape
    h1 = w1.shape[1]
    o = w2.shape[1]
    return pl.pallas_call(
        _mlp_body,
        out_shape=jax.ShapeDtypeStruct((n, o), jnp.float32),
        grid=(kdim // tk,),
        in_specs=[
            pl.BlockSpec((n, tk), lambda k: (0, k)),
            pl.BlockSpec((tk, h1), lambda k: (k, 0)),
            pl.BlockSpec((1, h1), lambda k: (0, 0)),
            pl.BlockSpec((h1, o), lambda k: (0, 0)),
            pl.BlockSpec((1, o), lambda k: (0, 0)),
        ],
        out_specs=pl.BlockSpec((n, o), lambda k: (0, 0)),
        scratch_shapes=[pltpu.VMEM((n, h1), jnp.float32)],
        compiler_params=pltpu.CompilerParams(
            dimension_semantics=("arbitrary",),
            vmem_limit_bytes=48 * 1024 * 1024),
    )(feats, w1, b1, w2, b2)


def kernel(x_nchw, c1w, c1b, c2w, c2b, c3w, c3b, f1w, f1b, f2w, f2b):
    x = jnp.transpose(x_nchw, (0, 2, 3, 1))
    x = _conv_layer(x, c1w, c1b, band=56)
    x = _conv_layer(x, c2w, c2b, band=112)
    x = _conv_layer(x, c3w, c3b, band=56)
    feats = x.reshape(x.shape[0], -1)
    return _mlp_head(feats, f1w, f1b, f2w, f2b, tk=18816)


# speedup vs baseline: 4.4297x; 3.6394x over previous
import functools

import jax
import jax.numpy as jnp
from jax.experimental import pallas as pl
from jax.experimental.pallas import tpu as pltpu


def _toep(w_taps, cin, cout, wp, w2, ph):
    w5 = w_taps.reshape(5, 5, cin, cout)
    w5p = jnp.pad(w5, ((0, 0), (0, 1), (0, 0), (0, 0)))
    wi = jnp.arange(wp)
    ji = jnp.arange(w2)
    u = wi[:, None] - 2 * ji[None, :] - ph
    uc = jnp.where((u >= 0) & (u < 5), u, 5)
    t = w5p[:, uc]
    return t.transpose(0, 1, 3, 2, 4).reshape(5, wp * cin, w2 * cout)


def _toep_pool_body(x_ref, te_ref, to_ref, b_ref, o_ref, acc_e, acc_o, *,
                    hout, nlanes):
    for kh in range(5):
        for t_ref, acc in ((te_ref, acc_e), (to_ref, acc_o)):
            z = jnp.dot(x_ref[...], t_ref[kh],
                        preferred_element_type=jnp.float32)
            zs = z[kh:kh + hout]
            if kh == 0:
                acc[...] = zs
            else:
                acc[...] += zs
    bias = b_ref[...]
    ae = jnp.maximum(acc_e[...] + bias, 0.0)
    ao = jnp.maximum(acc_o[...] + bias, 0.0)
    pw = jnp.maximum(ae, ao)
    o_ref[...] = pw.reshape(hout // 2, 2, nlanes).max(axis=1)


def _conv12(xf, w_taps, b_row, *, cin, cout):
    n, hp, kdim = xf.shape
    hout = hp - 4
    wp = kdim // cin
    w2 = (wp - 4) // 2
    nlanes = w2 * cout
    te = _toep(w_taps, cin, cout, wp, w2, 0)
    to = _toep(w_taps, cin, cout, wp, w2, 1)
    bt = jnp.tile(b_row, (1, w2))
    return pl.pallas_call(
        functools.partial(_toep_pool_body, hout=hout, nlanes=nlanes),
        out_shape=jax.ShapeDtypeStruct((n, hout // 2, nlanes), jnp.float32),
        grid=(n,),
        in_specs=[
            pl.BlockSpec((None, hp, kdim), lambda ni: (ni, 0, 0)),
            pl.BlockSpec(memory_space=pltpu.VMEM),
            pl.BlockSpec(memory_space=pltpu.VMEM),
            pl.BlockSpec(memory_space=pltpu.VMEM),
        ],
        out_specs=pl.BlockSpec((None, hout // 2, nlanes), lambda ni: (ni, 0, 0)),
        scratch_shapes=[pltpu.VMEM((hout, nlanes), jnp.float32),
                        pltpu.VMEM((hout, nlanes), jnp.float32)],
        compiler_params=pltpu.CompilerParams(
            dimension_semantics=("parallel",),
            vmem_limit_bytes=48 * 1024 * 1024),
    )(xf, te, to, bt)


_C3_IMGS = 8
_C3_ROWS = 64
_C3_NT = 4
_C3_K = 960
_C3_NL = 896


def _c3_body(x_ref, t_ref, b_ref, o_ref, acc_e, acc_o):
    m = _C3_IMGS * _C3_ROWS
    mv = m - _C3_ROWS + 56 + 4
    x2 = x_ref[...].reshape(m, _C3_K)
    for kh in range(5):
        for sl, acc in ((16, acc_e), (0, acc_o)):
            rhs = t_ref[kh, sl:sl + _C3_K, :]
            z = jnp.dot(x2, rhs, preferred_element_type=jnp.float32)
            zs = z[kh:kh + mv - 4]
            if kh == 0:
                acc[...] = zs
            else:
                acc[...] += zs
    bias = b_ref[...]
    ae = jnp.maximum(acc_e[...] + bias, 0.0)
    ao = jnp.maximum(acc_o[...] + bias, 0.0)
    pw = jnp.maximum(ae, ao)
    for i in range(_C3_IMGS):
        o_ref[i] = pw[i * _C3_ROWS:i * _C3_ROWS + 56].reshape(28, 2, _C3_NL).max(axis=1)


def _conv3(xf, w_taps, b_row):
    n = xf.shape[0]
    w5 = w_taps.reshape(5, 5, 16, 120)
    w5p = jnp.pad(w5, ((0, 0), (0, 1), (0, 0), (0, 8)))
    vi = jnp.arange(61)
    ji = jnp.arange(28)
    u = vi[:, None] - 1 - 2 * ji[None, :]
    uc = jnp.where((u >= 0) & (u < 5), u, 5)
    t3 = w5p[:, uc]
    t3 = t3.transpose(0, 1, 3, 2, 4).reshape(5, 976, 28 * 128)
    bt = jnp.tile(jnp.pad(b_row, ((0, 0), (0, 8))), (1, 7))
    return pl.pallas_call(
        _c3_body,
        out_shape=jax.ShapeDtypeStruct((n, 28, 28 * 128), jnp.float32),
        grid=(_C3_NT, n // _C3_IMGS),
        in_specs=[
            pl.BlockSpec((_C3_IMGS, _C3_ROWS, _C3_K), lambda t, ib: (ib, 0, 0)),
            pl.BlockSpec((5, 976, _C3_NL), lambda t, ib: (0, 0, t)),
            pl.BlockSpec((1, _C3_NL), lambda t, ib: (0, 0)),
        ],
        out_specs=pl.BlockSpec((_C3_IMGS, 28, _C3_NL), lambda t, ib: (ib, 0, t)),
        scratch_shapes=[pltpu.VMEM((504, _C3_NL), jnp.float32),
                        pltpu.VMEM((504, _C3_NL), jnp.float32)],
        compiler_params=pltpu.CompilerParams(
            dimension_semantics=("parallel", "arbitrary"),
            vmem_limit_bytes=48 * 1024 * 1024),
    )(xf, t3, bt)


def _mlp_body(x_ref, w1_ref, b1_ref, w2_ref, b2_ref, o_ref, acc_ref):
    k = pl.program_id(0)

    @pl.when(k == 0)
    def _():
        acc_ref[...] = jnp.zeros_like(acc_ref)

    acc_ref[...] += jnp.dot(x_ref[...], w1_ref[...],
                            preferred_element_type=jnp.float32)

    @pl.when(k == pl.num_programs(0) - 1)
    def _():
        h = jnp.maximum(acc_ref[...] + b1_ref[...], 0.0)
        o_ref[...] = jnp.dot(h, w2_ref[...],
                             preferred_element_type=jnp.float32) + b2_ref[...]


def _mlp_head(feats, w1, b1, w2, b2, *, tk):
    n, kdim = feats.shape
    h1 = w1.shape[1]
    o = w2.shape[1]
    return pl.pallas_call(
        _mlp_body,
        out_shape=jax.ShapeDtypeStruct((n, o), jnp.float32),
        grid=(kdim // tk,),
        in_specs=[
            pl.BlockSpec((n, tk), lambda k: (0, k)),
            pl.BlockSpec((tk, h1), lambda k: (k, 0)),
            pl.BlockSpec((1, h1), lambda k: (0, 0)),
            pl.BlockSpec((h1, o), lambda k: (0, 0)),
            pl.BlockSpec((1, o), lambda k: (0, 0)),
        ],
        out_specs=pl.BlockSpec((n, o), lambda k: (0, 0)),
        scratch_shapes=[pltpu.VMEM((n, h1), jnp.float32)],
        compiler_params=pltpu.CompilerParams(
            dimension_semantics=("arbitrary",),
            vmem_limit_bytes=48 * 1024 * 1024),
    )(feats, w1, b1, w2, b2)


def _pad_flat(y, wc):
    return jnp.pad(y, ((0, 0), (2, 2), (2 * wc, 2 * wc)))


def kernel(x_nchw, c1w, c1b, c2w, c2b, c3w, c3b, f1w, f1b, f2w, f2b):
    n = x_nchw.shape[0]
    x = jnp.transpose(x_nchw, (0, 2, 3, 1))
    x1 = jnp.pad(x, ((0, 0), (2, 2), (2, 2), (0, 0))).reshape(n, 228, 684)
    y1 = _conv12(x1, c1w, c1b, cin=3, cout=6)
    x2 = _pad_flat(y1, 6)
    y2 = _conv12(x2, c2w, c2b, cin=6, cout=16)
    x3 = jnp.pad(y2, ((0, 0), (2, 6), (32, 32)))
    y3 = _conv3(x3, c3w, c3b)
    feats = y3.reshape(n, 28, 28, 128)[:, :, :, :120].reshape(n, 94080)
    return _mlp_head(feats, f1w, f1b, f2w, f2b, tk=18816)


# speedup vs baseline: 6.2522x; 1.4115x over previous
import functools

import jax
import jax.numpy as jnp
from jax.experimental import pallas as pl
from jax.experimental.pallas import tpu as pltpu


def _toep(w_taps, cin, cout, wp, w2, ph):
    w5 = w_taps.reshape(5, 5, cin, cout)
    w5p = jnp.pad(w5, ((0, 0), (0, 1), (0, 0), (0, 0)))
    wi = jnp.arange(wp)
    ji = jnp.arange(w2)
    u = wi[:, None] - 2 * ji[None, :] - ph
    uc = jnp.where((u >= 0) & (u < 5), u, 5)
    t = w5p[:, uc]
    return t.transpose(0, 1, 3, 2, 4).reshape(5, wp * cin, w2 * cout)


def _c1_body(x_ref, t_ref, b_ref, o_ref, acc_e, acc_o):
    acc_e[...] = jnp.zeros_like(acc_e)
    acc_o[...] = jnp.zeros_like(acc_o)
    for kh in range(5):
        lo = max(0, 2 - kh)
        hi = min(224, 226 - kh)
        sl = lo + kh - 2
        for c in range(3):
            z = jnp.dot(x_ref[c], t_ref[c, kh],
                        preferred_element_type=jnp.float32)
            acc_e[lo:hi] += z[sl:sl + hi - lo, :768]
            acc_o[lo:hi] += z[sl:sl + hi - lo, 768:]
    bias = b_ref[...]
    ae = jnp.maximum(acc_e[...] + bias, 0.0)
    ao = jnp.maximum(acc_o[...] + bias, 0.0)
    pw = jnp.maximum(ae, ao)
    pooled = pw.reshape(112, 2, 768).max(axis=1)
    o_ref[...] = jnp.zeros_like(o_ref)
    o_ref[2:114, 12:684] = pooled[:, :672]


def _conv1(x_nchw, w_taps, b_row):
    n = x_nchw.shape[0]
    w5 = w_taps.reshape(5, 5, 3, 6)
    w5p = jnp.pad(w5, ((0, 0), (0, 1), (0, 0), (0, 0)))
    wi = jnp.arange(224)
    ji = jnp.arange(112)
    phases = []
    for ph in range(2):
        u = wi[:, None] + 2 - 2 * ji[None, :] - ph
        uc = jnp.where((u >= 0) & (u < 5), u, 5)
        t = w5p[:, uc]
        t = t.transpose(3, 0, 1, 2, 4).reshape(3, 5, 224, 672)
        phases.append(jnp.pad(t, ((0, 0),) * 3 + ((0, 96),)))
    t1 = jnp.concatenate(phases, axis=-1)
    bt = jnp.pad(jnp.tile(b_row, (1, 112)), ((0, 0), (0, 96)))
    return pl.pallas_call(
        _c1_body,
        out_shape=jax.ShapeDtypeStruct((n, 116, 696), jnp.float32),
        grid=(n,),
        in_specs=[
            pl.BlockSpec((None, 3, 224, 224), lambda ni: (ni, 0, 0, 0)),
            pl.BlockSpec(memory_space=pltpu.VMEM),
            pl.BlockSpec(memory_space=pltpu.VMEM),
        ],
        out_specs=pl.BlockSpec((None, 116, 696), lambda ni: (ni, 0, 0)),
        scratch_shapes=[pltpu.VMEM((224, 768), jnp.float32),
                        pltpu.VMEM((224, 768), jnp.float32)],
        compiler_params=pltpu.CompilerParams(
            dimension_semantics=("parallel",),
            vmem_limit_bytes=48 * 1024 * 1024),
    )(x_nchw, t1, bt)


def _c2_body(x_ref, te_ref, to_ref, b_ref, o_ref, acc_e, acc_o):
    for kh in range(5):
        for t_ref, acc in ((te_ref, acc_e), (to_ref, acc_o)):
            z = jnp.dot(x_ref[...], t_ref[kh],
                        preferred_element_type=jnp.float32)
            zs = z[kh:kh + 112]
            if kh == 0:
                acc[...] = zs
            else:
                acc[...] += zs
    bias = b_ref[...]
    ae = jnp.maximum(acc_e[...] + bias, 0.0)
    ao = jnp.maximum(acc_o[...] + bias, 0.0)
    pw = jnp.maximum(ae, ao)
    pooled = pw.reshape(56, 2, 896).max(axis=1)
    o_ref[...] = jnp.zeros_like(o_ref)
    o_ref[2:58, 32:928] = pooled


def _conv2(xf, w_taps, b_row):
    n = xf.shape[0]
    te = _toep(w_taps, 6, 16, 116, 56, 0)
    to = _toep(w_taps, 6, 16, 116, 56, 1)
    bt = jnp.tile(b_row, (1, 56))
    return pl.pallas_call(
        _c2_body,
        out_shape=jax.ShapeDtypeStruct((n, 64, 960), jnp.float32),
        grid=(n,),
        in_specs=[
            pl.BlockSpec((None, 116, 696), lambda ni: (ni, 0, 0)),
            pl.BlockSpec(memory_space=pltpu.VMEM),
            pl.BlockSpec(memory_space=pltpu.VMEM),
            pl.BlockSpec(memory_space=pltpu.VMEM),
        ],
        out_specs=pl.BlockSpec((None, 64, 960), lambda ni: (ni, 0, 0)),
        scratch_shapes=[pltpu.VMEM((112, 896), jnp.float32),
                        pltpu.VMEM((112, 896), jnp.float32)],
        compiler_params=pltpu.CompilerParams(
            dimension_semantics=("parallel",),
            vmem_limit_bytes=48 * 1024 * 1024),
    )(xf, te, to, bt)


_C3_IMGS = 8
_C3_ROWS = 64
_C3_NT = 4
_C3_K = 960
_C3_NL = 896


def _c3_body(x_ref, t_ref, b_ref, o_ref, acc_e, acc_o):
    m = _C3_IMGS * _C3_ROWS
    mv = m - _C3_ROWS + 56 + 4
    x2 = x_ref[...].reshape(m, _C3_K)
    for kh in range(5):
        for sl, acc in ((16, acc_e), (0, acc_o)):
            rhs = t_ref[kh, sl:sl + _C3_K, :]
            z = jnp.dot(x2, rhs, preferred_element_type=jnp.float32)
            zs = z[kh:kh + mv - 4]
            if kh == 0:
                acc[...] = zs
            else:
                acc[...] += zs
    bias = b_ref[...]
    ae = jnp.maximum(acc_e[...] + bias, 0.0)
    ao = jnp.maximum(acc_o[...] + bias, 0.0)
    pw = jnp.maximum(ae, ao)
    for i in range(_C3_IMGS):
        o_ref[i] = pw[i * _C3_ROWS:i * _C3_ROWS + 56].reshape(28, 2, _C3_NL).max(axis=1)


def _conv3(xf, w_taps, b_row):
    n = xf.shape[0]
    w5 = w_taps.reshape(5, 5, 16, 120)
    w5p = jnp.pad(w5, ((0, 0), (0, 1), (0, 0), (0, 8)))
    vi = jnp.arange(61)
    ji = jnp.arange(28)
    u = vi[:, None] - 1 - 2 * ji[None, :]
    uc = jnp.where((u >= 0) & (u < 5), u, 5)
    t3 = w5p[:, uc]
    t3 = t3.transpose(0, 1, 3, 2, 4).reshape(5, 976, 28 * 128)
    bt = jnp.tile(jnp.pad(b_row, ((0, 0), (0, 8))), (1, 7))
    return pl.pallas_call(
        _c3_body,
        out_shape=jax.ShapeDtypeStruct((n, 28, 28 * 128), jnp.float32),
        grid=(_C3_NT, n // _C3_IMGS),
        in_specs=[
            pl.BlockSpec((_C3_IMGS, _C3_ROWS, _C3_K), lambda t, ib: (ib, 0, 0)),
            pl.BlockSpec((5, 976, _C3_NL), lambda t, ib: (0, 0, t)),
            pl.BlockSpec((1, _C3_NL), lambda t, ib: (0, 0)),
        ],
        out_specs=pl.BlockSpec((_C3_IMGS, 28, _C3_NL), lambda t, ib: (ib, 0, t)),
        scratch_shapes=[pltpu.VMEM((504, _C3_NL), jnp.float32),
                        pltpu.VMEM((504, _C3_NL), jnp.float32)],
        compiler_params=pltpu.CompilerParams(
            dimension_semantics=("parallel", "arbitrary"),
            vmem_limit_bytes=48 * 1024 * 1024),
    )(xf, t3, bt)


def _mlp_body(x_ref, w1_ref, b1_ref, w2_ref, b2_ref, o_ref, acc_ref):
    k = pl.program_id(0)

    @pl.when(k == 0)
    def _():
        acc_ref[...] = jnp.zeros_like(acc_ref)

    acc_ref[...] += jnp.dot(x_ref[...], w1_ref[...],
                            preferred_element_type=jnp.float32)

    @pl.when(k == pl.num_programs(0) - 1)
    def _():
        h = jnp.maximum(acc_ref[...] + b1_ref[...], 0.0)
        o_ref[...] = jnp.dot(h, w2_ref[...],
                             preferred_element_type=jnp.float32) + b2_ref[...]


def _mlp_head(feats, w1, b1, w2, b2, *, tk):
    n, kdim = feats.shape
    h1 = w1.shape[1]
    o = w2.shape[1]
    return pl.pallas_call(
        _mlp_body,
        out_shape=jax.ShapeDtypeStruct((n, o), jnp.float32),
        grid=(kdim // tk,),
        in_specs=[
            pl.BlockSpec((n, tk), lambda k: (0, k)),
            pl.BlockSpec((tk, h1), lambda k: (k, 0)),
            pl.BlockSpec((1, h1), lambda k: (0, 0)),
            pl.BlockSpec((h1, o), lambda k: (0, 0)),
            pl.BlockSpec((1, o), lambda k: (0, 0)),
        ],
        out_specs=pl.BlockSpec((n, o), lambda k: (0, 0)),
        scratch_shapes=[pltpu.VMEM((n, h1), jnp.float32)],
        compiler_params=pltpu.CompilerParams(
            dimension_semantics=("arbitrary",),
            vmem_limit_bytes=48 * 1024 * 1024),
    )(feats, w1, b1, w2, b2)


def _pad_flat(y, wc):
    return jnp.pad(y, ((0, 0), (2, 2), (2 * wc, 2 * wc)))


def kernel(x_nchw, c1w, c1b, c2w, c2b, c3w, c3b, f1w, f1b, f2w, f2b):
    n = x_nchw.shape[0]
    x2 = _conv1(x_nchw, c1w, c1b)
    x3 = _conv2(x2, c2w, c2b)
    y3 = _conv3(x3, c3w, c3b)
    feats = y3.reshape(n, 28, 28, 128)[:, :, :, :120].reshape(n, 94080)
    return _mlp_head(feats, f1w, f1b, f2w, f2b, tk=18816)


# speedup vs baseline: 9.7016x; 1.5517x over previous
import functools

import jax
import jax.numpy as jnp
from jax.experimental import pallas as pl
from jax.experimental.pallas import tpu as pltpu


def _band_mask(wp, w2, jpad, ph):
    ui = jnp.arange(5)[:, None, None]
    wi = jnp.arange(wp)[None, :, None]
    ji = jnp.arange(jpad)[None, None, :]
    return ((wi - 2 * ji - ph == ui) & (ji < w2)).astype(jnp.float32)


def _toep(w_taps, cin, cout, wp, w2, jpad):
    w5 = w_taps.reshape(5, 5, cin, cout)
    d = jnp.stack([_band_mask(wp, w2, jpad, ph) for ph in range(2)], axis=2)
    t = (w5[:, :, None, :, None, None, :] *
         d[None, :, :, None, :, :, None]).sum(axis=1)
    return t.reshape(5, wp * cin, 2 * jpad * cout)


def _c1_body(x_ref, t_ref, b_ref, o_ref, acc_e, acc_o):
    acc_e[...] = jnp.zeros_like(acc_e)
    acc_o[...] = jnp.zeros_like(acc_o)
    for kh in range(5):
        lo = max(0, 2 - kh)
        hi = min(224, 226 - kh)
        sl = lo + kh - 2
        for c in range(3):
            z = jnp.dot(x_ref[c], t_ref[c, kh],
                        preferred_element_type=jnp.float32)
            acc_e[lo:hi] += z[sl:sl + hi - lo, :768]
            acc_o[lo:hi] += z[sl:sl + hi - lo, 768:]
    bias = b_ref[...]
    ae = jnp.maximum(acc_e[...] + bias, 0.0)
    ao = jnp.maximum(acc_o[...] + bias, 0.0)
    pw = jnp.maximum(ae, ao)
    pooled = pw.reshape(112, 2, 768).max(axis=1)
    o_ref[...] = jnp.zeros_like(o_ref)
    o_ref[2:114, 12:684] = pooled[:, :672]


def _conv1(x_nchw, w_taps, b_row):
    n = x_nchw.shape[0]
    w5c = w_taps.reshape(5, 5, 3, 6).transpose(2, 0, 1, 3)
    ui = jnp.arange(5)[:, None, None, None]
    wi = jnp.arange(224)[None, :, None, None]
    phi = jnp.arange(2)[None, None, :, None]
    ji = jnp.arange(128)[None, None, None, :]
    d = ((wi + 2 - 2 * ji - phi == ui) & (ji < 112)).astype(jnp.float32)
    t1 = (w5c[:, :, :, None, None, None, :] *
          d[None, None, :, :, :, :, None]).sum(axis=2)
    t1 = t1.reshape(3, 5, 224, 1536)
    bt = jnp.pad(jnp.tile(b_row, (1, 112)), ((0, 0), (0, 96)))
    return pl.pallas_call(
        _c1_body,
        out_shape=jax.ShapeDtypeStruct((n, 116, 696), jnp.float32),
        grid=(n,),
        in_specs=[
            pl.BlockSpec((None, 3, 224, 224), lambda ni: (ni, 0, 0, 0)),
            pl.BlockSpec(memory_space=pltpu.VMEM),
            pl.BlockSpec(memory_space=pltpu.VMEM),
        ],
        out_specs=pl.BlockSpec((None, 116, 696), lambda ni: (ni, 0, 0)),
        scratch_shapes=[pltpu.VMEM((224, 768), jnp.float32),
                        pltpu.VMEM((224, 768), jnp.float32)],
        compiler_params=pltpu.CompilerParams(
            dimension_semantics=("parallel",),
            vmem_limit_bytes=48 * 1024 * 1024),
    )(x_nchw, t1, bt)


def _c2_body(x_ref, t_ref, b_ref, o_ref, acc):
    for kh in range(5):
        z = jnp.dot(x_ref[...], t_ref[kh],
                    preferred_element_type=jnp.float32)
        zs = z[kh:kh + 112]
        if kh == 0:
            acc[...] = zs
        else:
            acc[...] += zs
    a = jnp.maximum(acc[...] + b_ref[...], 0.0)
    pw = jnp.maximum(a[:, :896], a[:, 896:])
    pooled = pw.reshape(56, 2, 896).max(axis=1)
    o_ref[...] = jnp.zeros_like(o_ref)
    o_ref[2:58, 32:928] = pooled


def _conv2(xf, w_taps, b_row):
    n = xf.shape[0]
    t2 = _toep(w_taps, 6, 16, 116, 56, 56)
    bt = jnp.tile(b_row, (1, 112))
    return pl.pallas_call(
        _c2_body,
        out_shape=jax.ShapeDtypeStruct((n, 64, 960), jnp.float32),
        grid=(n,),
        in_specs=[
            pl.BlockSpec((None, 116, 696), lambda ni: (ni, 0, 0)),
            pl.BlockSpec(memory_space=pltpu.VMEM),
            pl.BlockSpec(memory_space=pltpu.VMEM),
        ],
        out_specs=pl.BlockSpec((None, 64, 960), lambda ni: (ni, 0, 0)),
        scratch_shapes=[pltpu.VMEM((112, 1792), jnp.float32)],
        compiler_params=pltpu.CompilerParams(
            dimension_semantics=("parallel",),
            vmem_limit_bytes=48 * 1024 * 1024),
    )(xf, t2, bt)


_C3_IMGS = 8
_C3_ROWS = 64
_C3_NT = 4
_C3_K = 960
_C3_NL = 896


def _c3_body(x_ref, t_ref, b_ref, o_ref, acc_e, acc_o):
    m = _C3_IMGS * _C3_ROWS
    mv = m - _C3_ROWS + 56 + 4
    x2 = x_ref[...].reshape(m, _C3_K)
    for kh in range(5):
        for sl, acc in ((16, acc_e), (0, acc_o)):
            rhs = t_ref[kh, sl:sl + _C3_K, :]
            z = jnp.dot(x2, rhs, preferred_element_type=jnp.float32)
            zs = z[kh:kh + mv - 4]
            if kh == 0:
                acc[...] = zs
            else:
                acc[...] += zs
    bias = b_ref[...]
    ae = jnp.maximum(acc_e[...] + bias, 0.0)
    ao = jnp.maximum(acc_o[...] + bias, 0.0)
    pw = jnp.maximum(ae, ao)
    for i in range(_C3_IMGS):
        o_ref[i] = pw[i * _C3_ROWS:i * _C3_ROWS + 56].reshape(28, 2, _C3_NL).max(axis=1)


def _conv3(xf, w_taps, b_row):
    n = xf.shape[0]
    w5p = jnp.pad(w_taps.reshape(5, 5, 16, 120),
                  ((0, 0), (0, 0), (0, 0), (0, 8)))
    ui = jnp.arange(5)[:, None, None]
    vi = jnp.arange(61)[None, :, None]
    ji = jnp.arange(28)[None, None, :]
    d3 = (vi - 1 - 2 * ji == ui).astype(jnp.float32)
    t3 = (w5p[:, :, None, :, None, :] *
          d3[None, :, :, None, :, None]).sum(axis=1)
    t3 = t3.reshape(5, 976, 28 * 128)
    bt = jnp.tile(jnp.pad(b_row, ((0, 0), (0, 8))), (1, 7))
    return pl.pallas_call(
        _c3_body,
        out_shape=jax.ShapeDtypeStruct((n, 28, 28 * 128), jnp.float32),
        grid=(_C3_NT, n // _C3_IMGS),
        in_specs=[
            pl.BlockSpec((_C3_IMGS, _C3_ROWS, _C3_K), lambda t, ib: (ib, 0, 0)),
            pl.BlockSpec((5, 976, _C3_NL), lambda t, ib: (0, 0, t)),
            pl.BlockSpec((1, _C3_NL), lambda t, ib: (0, 0)),
        ],
        out_specs=pl.BlockSpec((_C3_IMGS, 28, _C3_NL), lambda t, ib: (ib, 0, t)),
        scratch_shapes=[pltpu.VMEM((504, _C3_NL), jnp.float32),
                        pltpu.VMEM((504, _C3_NL), jnp.float32)],
        compiler_params=pltpu.CompilerParams(
            dimension_semantics=("parallel", "arbitrary"),
            vmem_limit_bytes=48 * 1024 * 1024),
    )(xf, t3, bt)


def _mlp_body(x_ref, w1_ref, b1_ref, w2_ref, b2_ref, o_ref, acc_ref):
    k = pl.program_id(0)

    @pl.when(k == 0)
    def _():
        acc_ref[...] = jnp.zeros_like(acc_ref)

    acc_ref[...] += jnp.dot(x_ref[...], w1_ref[...],
                            preferred_element_type=jnp.float32)

    @pl.when(k == pl.num_programs(0) - 1)
    def _():
        h = jnp.maximum(acc_ref[...] + b1_ref[...], 0.0)
        o_ref[...] = jnp.dot(h, w2_ref[...],
                             preferred_element_type=jnp.float32) + b2_ref[...]


def _mlp_head(feats, w1, b1, w2, b2, *, tk):
    n, kdim = feats.shape
    h1 = w1.shape[1]
    o = w2.shape[1]
    return pl.pallas_call(
        _mlp_body,
        out_shape=jax.ShapeDtypeStruct((n, o), jnp.float32),
        grid=(kdim // tk,),
        in_specs=[
            pl.BlockSpec((n, tk), lambda k: (0, k)),
            pl.BlockSpec((tk, h1), lambda k: (k, 0)),
            pl.BlockSpec((1, h1), lambda k: (0, 0)),
            pl.BlockSpec((h1, o), lambda k: (0, 0)),
            pl.BlockSpec((1, o), lambda k: (0, 0)),
        ],
        out_specs=pl.BlockSpec((n, o), lambda k: (0, 0)),
        scratch_shapes=[pltpu.VMEM((n, h1), jnp.float32)],
        compiler_params=pltpu.CompilerParams(
            dimension_semantics=("arbitrary",),
            vmem_limit_bytes=48 * 1024 * 1024),
    )(feats, w1, b1, w2, b2)


def _pad_flat(y, wc):
    return jnp.pad(y, ((0, 0), (2, 2), (2 * wc, 2 * wc)))


def kernel(x_nchw, c1w, c1b, c2w, c2b, c3w, c3b, f1w, f1b, f2w, f2b):
    n = x_nchw.shape[0]
    x2 = _conv1(x_nchw, c1w, c1b)
    x3 = _conv2(x2, c2w, c2b)
    y3 = _conv3(x3, c3w, c3b)
    feats = y3.reshape(n, 28, 28, 128)[:, :, :, :120].reshape(n, 94080)
    return _mlp_head(feats, f1w, f1b, f2w, f2b, tk=18816)


# speedup vs baseline: 13.0764x; 1.3479x over previous
import functools

import jax
import jax.numpy as jnp
from jax.experimental import pallas as pl
from jax.experimental.pallas import tpu as pltpu


def _t2_build_body(w_ref, o_ref):
    o_ref[...] = jnp.zeros_like(o_ref)
    blk = w_ref[...]
    for ph in range(2):
        for j in range(56):
            rs = 6 * (2 * j + ph)
            ls = 896 * ph + 16 * j
            o_ref[:, rs:rs + 30, ls:ls + 16] = blk


def _toep2(w_taps):
    w5 = w_taps.reshape(5, 30, 16)
    return pl.pallas_call(
        _t2_build_body,
        out_shape=jax.ShapeDtypeStruct((5, 696, 1792), jnp.float32),
        in_specs=[pl.BlockSpec(memory_space=pltpu.VMEM)],
        out_specs=pl.BlockSpec(memory_space=pltpu.VMEM),
        compiler_params=pltpu.CompilerParams(
            vmem_limit_bytes=48 * 1024 * 1024),
    )(w5)


def _t1_build_body(w_ref, o_ref):
    o_ref[...] = jnp.zeros_like(o_ref)
    blk = w_ref[...]
    for ph in range(2):
        for j in range(112):
            w0 = max(0, 2 * j + ph - 2)
            w1 = min(224, 2 * j + ph + 3)
            u0 = w0 - (2 * j + ph - 2)
            ls = 768 * ph + 6 * j
            o_ref[:, :, w0:w1, ls:ls + 6] = blk[:, :, u0:u0 + w1 - w0, :]


def _toep1(w_taps):
    w5c = w_taps.reshape(5, 5, 3, 6).transpose(2, 0, 1, 3)
    return pl.pallas_call(
        _t1_build_body,
        out_shape=jax.ShapeDtypeStruct((3, 5, 224, 1536), jnp.float32),
        in_specs=[pl.BlockSpec(memory_space=pltpu.VMEM)],
        out_specs=pl.BlockSpec(memory_space=pltpu.VMEM),
        compiler_params=pltpu.CompilerParams(
            vmem_limit_bytes=48 * 1024 * 1024),
    )(w5c)


def _t3_build_body(w_ref, o_ref):
    j = pl.program_id(0)
    o_ref[...] = jnp.zeros_like(o_ref)
    base = pl.multiple_of(16 * (2 * j + 1), 16)
    o_ref[:, pl.ds(base, 80), :] = w_ref[...]


def _toep3(w_taps):
    w5p = jnp.pad(w_taps.reshape(5, 5, 16, 120),
                  ((0, 0), (0, 0), (0, 0), (0, 8))).reshape(5, 80, 128)
    return pl.pallas_call(
        _t3_build_body,
        out_shape=jax.ShapeDtypeStruct((5, 976, 3584), jnp.float32),
        grid=(28,),
        in_specs=[pl.BlockSpec(memory_space=pltpu.VMEM)],
        out_specs=pl.BlockSpec((5, 976, 128), lambda j: (0, 0, j)),
        compiler_params=pltpu.CompilerParams(
            dimension_semantics=("arbitrary",),
            vmem_limit_bytes=48 * 1024 * 1024),
    )(w5p)


def _c1_body(x_ref, t_ref, b_ref, o_ref, acc_e, acc_o):
    acc_e[...] = jnp.zeros_like(acc_e)
    acc_o[...] = jnp.zeros_like(acc_o)
    for kh in range(5):
        lo = max(0, 2 - kh)
        hi = min(224, 226 - kh)
        sl = lo + kh - 2
        for c in range(3):
            z = jnp.dot(x_ref[c], t_ref[c, kh],
                        preferred_element_type=jnp.float32)
            acc_e[lo:hi] += z[sl:sl + hi - lo, :768]
            acc_o[lo:hi] += z[sl:sl + hi - lo, 768:]
    bias = b_ref[...]
    ae = jnp.maximum(acc_e[...] + bias, 0.0)
    ao = jnp.maximum(acc_o[...] + bias, 0.0)
    pw = jnp.maximum(ae, ao)
    pooled = pw.reshape(112, 2, 768).max(axis=1)
    o_ref[...] = jnp.zeros_like(o_ref)
    o_ref[2:114, 12:684] = pooled[:, :672]


def _conv1(x_nchw, w_taps, b_row):
    n = x_nchw.shape[0]
    t1 = _toep1(w_taps)
    bt = jnp.pad(jnp.tile(b_row, (1, 112)), ((0, 0), (0, 96)))
    return pl.pallas_call(
        _c1_body,
        out_shape=jax.ShapeDtypeStruct((n, 116, 696), jnp.float32),
        grid=(n,),
        in_specs=[
            pl.BlockSpec((None, 3, 224, 224), lambda ni: (ni, 0, 0, 0)),
            pl.BlockSpec(memory_space=pltpu.VMEM),
            pl.BlockSpec(memory_space=pltpu.VMEM),
        ],
        out_specs=pl.BlockSpec((None, 116, 696), lambda ni: (ni, 0, 0)),
        scratch_shapes=[pltpu.VMEM((224, 768), jnp.float32),
                        pltpu.VMEM((224, 768), jnp.float32)],
        compiler_params=pltpu.CompilerParams(
            dimension_semantics=("parallel",),
            vmem_limit_bytes=48 * 1024 * 1024),
    )(x_nchw, t1, bt)


def _c2_body(x_ref, t_ref, b_ref, o_ref, acc):
    for kh in range(5):
        z = jnp.dot(x_ref[...], t_ref[kh],
                    preferred_element_type=jnp.float32)
        zs = z[kh:kh + 112]
        if kh == 0:
            acc[...] = zs
        else:
            acc[...] += zs
    a = jnp.maximum(acc[...] + b_ref[...], 0.0)
    pw = jnp.maximum(a[:, :896], a[:, 896:])
    pooled = pw.reshape(56, 2, 896).max(axis=1)
    o_ref[...] = jnp.zeros_like(o_ref)
    o_ref[2:58, 32:928] = pooled


def _conv2(xf, w_taps, b_row):
    n = xf.shape[0]
    t2 = _toep2(w_taps)
    bt = jnp.tile(b_row, (1, 112))
    return pl.pallas_call(
        _c2_body,
        out_shape=jax.ShapeDtypeStruct((n, 64, 960), jnp.float32),
        grid=(n,),
        in_specs=[
            pl.BlockSpec((None, 116, 696), lambda ni: (ni, 0, 0)),
            pl.BlockSpec(memory_space=pltpu.VMEM),
            pl.BlockSpec(memory_space=pltpu.VMEM),
        ],
        out_specs=pl.BlockSpec((None, 64, 960), lambda ni: (ni, 0, 0)),
        scratch_shapes=[pltpu.VMEM((112, 1792), jnp.float32)],
        compiler_params=pltpu.CompilerParams(
            dimension_semantics=("parallel",),
            vmem_limit_bytes=48 * 1024 * 1024),
    )(xf, t2, bt)


_C3_IMGS = 8
_C3_ROWS = 64
_C3_NT = 4
_C3_K = 960
_C3_NL = 896


def _c3_body(x_ref, t_ref, b_ref, o_ref, acc_e, acc_o):
    m = _C3_IMGS * _C3_ROWS
    mv = m - _C3_ROWS + 56 + 4
    x2 = x_ref[...].reshape(m, _C3_K)
    for kh in range(5):
        for sl, acc in ((16, acc_e), (0, acc_o)):
            rhs = t_ref[kh, sl:sl + _C3_K, :]
            z = jnp.dot(x2, rhs, preferred_element_type=jnp.float32)
            zs = z[kh:kh + mv - 4]
            if kh == 0:
                acc[...] = zs
            else:
                acc[...] += zs
    bias = b_ref[...]
    ae = jnp.maximum(acc_e[...] + bias, 0.0)
    ao = jnp.maximum(acc_o[...] + bias, 0.0)
    pw = jnp.maximum(ae, ao)
    for i in range(_C3_IMGS):
        o_ref[i] = pw[i * _C3_ROWS:i * _C3_ROWS + 56].reshape(28, 2, _C3_NL).max(axis=1)


def _conv3(xf, w_taps, b_row):
    n = xf.shape[0]
    t3 = _toep3(w_taps)
    bt = jnp.tile(jnp.pad(b_row, ((0, 0), (0, 8))), (1, 7))
    return pl.pallas_call(
        _c3_body,
        out_shape=jax.ShapeDtypeStruct((n, 28, 28 * 128), jnp.float32),
        grid=(_C3_NT, n // _C3_IMGS),
        in_specs=[
            pl.BlockSpec((_C3_IMGS, _C3_ROWS, _C3_K), lambda t, ib: (ib, 0, 0)),
            pl.BlockSpec((5, 976, _C3_NL), lambda t, ib: (0, 0, t)),
            pl.BlockSpec((1, _C3_NL), lambda t, ib: (0, 0)),
        ],
        out_specs=pl.BlockSpec((_C3_IMGS, 28, _C3_NL), lambda t, ib: (ib, 0, t)),
        scratch_shapes=[pltpu.VMEM((504, _C3_NL), jnp.float32),
                        pltpu.VMEM((504, _C3_NL), jnp.float32)],
        compiler_params=pltpu.CompilerParams(
            dimension_semantics=("parallel", "arbitrary"),
            vmem_limit_bytes=48 * 1024 * 1024),
    )(xf, t3, bt)


def _mlp_body(x_ref, w1_ref, b1_ref, w2_ref, b2_ref, o_ref, acc_ref):
    k = pl.program_id(0)

    @pl.when(k == 0)
    def _():
        acc_ref[...] = jnp.zeros_like(acc_ref)

    acc_ref[...] += jnp.dot(x_ref[...], w1_ref[...],
                            preferred_element_type=jnp.float32)

    @pl.when(k == pl.num_programs(0) - 1)
    def _():
        h = jnp.maximum(acc_ref[...] + b1_ref[...], 0.0)
        o_ref[...] = jnp.dot(h, w2_ref[...],
                             preferred_element_type=jnp.float32) + b2_ref[...]


def _mlp_head(feats, w1, b1, w2, b2, *, tk):
    n, kdim = feats.shape
    h1 = w1.shape[1]
    o = w2.shape[1]
    return pl.pallas_call(
        _mlp_body,
        out_shape=jax.ShapeDtypeStruct((n, o), jnp.float32),
        grid=(kdim // tk,),
        in_specs=[
            pl.BlockSpec((n, tk), lambda k: (0, k)),
            pl.BlockSpec((tk, h1), lambda k: (k, 0)),
            pl.BlockSpec((1, h1), lambda k: (0, 0)),
            pl.BlockSpec((h1, o), lambda k: (0, 0)),
            pl.BlockSpec((1, o), lambda k: (0, 0)),
        ],
        out_specs=pl.BlockSpec((n, o), lambda k: (0, 0)),
        scratch_shapes=[pltpu.VMEM((n, h1), jnp.float32)],
        compiler_params=pltpu.CompilerParams(
            dimension_semantics=("arbitrary",),
            vmem_limit_bytes=48 * 1024 * 1024),
    )(feats, w1, b1, w2, b2)


def _pad_flat(y, wc):
    return jnp.pad(y, ((0, 0), (2, 2), (2 * wc, 2 * wc)))


def kernel(x_nchw, c1w, c1b, c2w, c2b, c3w, c3b, f1w, f1b, f2w, f2b):
    n = x_nchw.shape[0]
    x2 = _conv1(x_nchw, c1w, c1b)
    x3 = _conv2(x2, c2w, c2b)
    y3 = _conv3(x3, c3w, c3b)
    feats = y3.reshape(n, 28, 28, 128)[:, :, :, :120].reshape(n, 94080)
    return _mlp_head(feats, f1w, f1b, f2w, f2b, tk=18816)


# speedup vs baseline: 13.3590x; 1.0216x over previous
import functools

import jax
import jax.numpy as jnp
from jax.experimental import pallas as pl
from jax.experimental.pallas import tpu as pltpu


def _t2_build_body(w_ref, o_ref):
    o_ref[...] = jnp.zeros_like(o_ref)
    blk = w_ref[...]
    for ph in range(2):
        for j in range(56):
            rs = 6 * (2 * j + ph)
            ls = 896 * ph + 16 * j
            o_ref[:, rs:rs + 30, ls:ls + 16] = blk


def _toep2(w_taps):
    w5 = w_taps.reshape(5, 30, 16)
    return pl.pallas_call(
        _t2_build_body,
        out_shape=jax.ShapeDtypeStruct((5, 696, 1792), jnp.float32),
        in_specs=[pl.BlockSpec(memory_space=pltpu.VMEM)],
        out_specs=pl.BlockSpec(memory_space=pltpu.VMEM),
        compiler_params=pltpu.CompilerParams(
            vmem_limit_bytes=48 * 1024 * 1024),
    )(w5)


def _t1_build_body(w_ref, o_ref):
    o_ref[...] = jnp.zeros_like(o_ref)
    blk = w_ref[...]
    for ph in range(2):
        for j in range(112):
            w0 = max(0, 2 * j + ph - 2)
            w1 = min(224, 2 * j + ph + 3)
            u0 = w0 - (2 * j + ph - 2)
            ls = 768 * ph + 6 * j
            o_ref[:, :, w0:w1, ls:ls + 6] = blk[:, :, u0:u0 + w1 - w0, :]


def _toep1(w_taps):
    w5c = w_taps.reshape(5, 5, 3, 6).transpose(2, 0, 1, 3)
    return pl.pallas_call(
        _t1_build_body,
        out_shape=jax.ShapeDtypeStruct((3, 5, 224, 1536), jnp.float32),
        in_specs=[pl.BlockSpec(memory_space=pltpu.VMEM)],
        out_specs=pl.BlockSpec(memory_space=pltpu.VMEM),
        compiler_params=pltpu.CompilerParams(
            vmem_limit_bytes=48 * 1024 * 1024),
    )(w5c)


def _t3_build_body(w_ref, o_ref):
    j = pl.program_id(0)
    o_ref[...] = jnp.zeros_like(o_ref)
    base = pl.multiple_of(16 * (2 * j + 1), 16)
    o_ref[:, pl.ds(base, 80), :] = w_ref[...]


def _toep3(w_taps):
    w5p = jnp.pad(w_taps.reshape(5, 5, 16, 120),
                  ((0, 0), (0, 0), (0, 0), (0, 8))).reshape(5, 80, 128)
    return pl.pallas_call(
        _t3_build_body,
        out_shape=jax.ShapeDtypeStruct((5, 976, 3584), jnp.float32),
        grid=(28,),
        in_specs=[pl.BlockSpec(memory_space=pltpu.VMEM)],
        out_specs=pl.BlockSpec((5, 976, 128), lambda j: (0, 0, j)),
        compiler_params=pltpu.CompilerParams(
            dimension_semantics=("parallel",),
            vmem_limit_bytes=48 * 1024 * 1024),
    )(w5p)


def _c1_body(x_ref, t_ref, b_ref, o_ref, acc_e, acc_o):
    acc_e[...] = jnp.zeros_like(acc_e)
    acc_o[...] = jnp.zeros_like(acc_o)
    for kh in range(5):
        lo = max(0, 2 - kh)
        hi = min(224, 226 - kh)
        sl = lo + kh - 2
        for c in range(3):
            z = jnp.dot(x_ref[c], t_ref[c, kh],
                        preferred_element_type=jnp.float32)
            acc_e[lo:hi] += z[sl:sl + hi - lo, :768]
            acc_o[lo:hi] += z[sl:sl + hi - lo, 768:]
    bias = b_ref[...]
    ae = jnp.maximum(acc_e[...] + bias, 0.0)
    ao = jnp.maximum(acc_o[...] + bias, 0.0)
    pw = jnp.maximum(ae, ao)
    pooled = pw.reshape(112, 2, 768).max(axis=1)
    o_ref[...] = jnp.zeros_like(o_ref)
    o_ref[2:114, 12:684] = pooled[:, :672]


def _conv1(x_nchw, w_taps, b_row):
    n = x_nchw.shape[0]
    t1 = _toep1(w_taps)
    bt = jnp.pad(jnp.tile(b_row, (1, 112)), ((0, 0), (0, 96)))
    return pl.pallas_call(
        _c1_body,
        out_shape=jax.ShapeDtypeStruct((n, 116, 696), jnp.float32),
        grid=(n,),
        in_specs=[
            pl.BlockSpec((None, 3, 224, 224), lambda ni: (ni, 0, 0, 0)),
            pl.BlockSpec(memory_space=pltpu.VMEM),
            pl.BlockSpec(memory_space=pltpu.VMEM),
        ],
        out_specs=pl.BlockSpec((None, 116, 696), lambda ni: (ni, 0, 0)),
        scratch_shapes=[pltpu.VMEM((224, 768), jnp.float32),
                        pltpu.VMEM((224, 768), jnp.float32)],
        compiler_params=pltpu.CompilerParams(
            dimension_semantics=("parallel",),
            vmem_limit_bytes=48 * 1024 * 1024),
    )(x_nchw, t1, bt)


def _c2_body(x_ref, t_ref, b_ref, o_ref, acc):
    for kh in range(5):
        z = jnp.dot(x_ref[kh:kh + 112, :], t_ref[kh],
                    preferred_element_type=jnp.float32)
        if kh == 0:
            acc[...] = z
        else:
            acc[...] += z
    a = jnp.maximum(acc[...] + b_ref[...], 0.0)
    pw = jnp.maximum(a[:, :896], a[:, 896:])
    pooled = pw.reshape(56, 2, 896).max(axis=1)
    o_ref[...] = jnp.zeros_like(o_ref)
    o_ref[2:58, 32:928] = pooled


def _conv2(xf, w_taps, b_row):
    n = xf.shape[0]
    t2 = _toep2(w_taps)
    bt = jnp.tile(b_row, (1, 112))
    return pl.pallas_call(
        _c2_body,
        out_shape=jax.ShapeDtypeStruct((n, 64, 960), jnp.float32),
        grid=(n,),
        in_specs=[
            pl.BlockSpec((None, 116, 696), lambda ni: (ni, 0, 0)),
            pl.BlockSpec(memory_space=pltpu.VMEM),
            pl.BlockSpec(memory_space=pltpu.VMEM),
        ],
        out_specs=pl.BlockSpec((None, 64, 960), lambda ni: (ni, 0, 0)),
        scratch_shapes=[pltpu.VMEM((112, 1792), jnp.float32)],
        compiler_params=pltpu.CompilerParams(
            dimension_semantics=("parallel",),
            vmem_limit_bytes=48 * 1024 * 1024),
    )(xf, t2, bt)


_C3_IMGS = 8
_C3_ROWS = 64
_C3_NT = 4
_C3_K = 960
_C3_NL = 896


def _c3_body(x_ref, t_ref, b_ref, o_ref, acc_e, acc_o):
    m = _C3_IMGS * _C3_ROWS
    mv = m - _C3_ROWS + 56 + 4
    x2 = x_ref[...].reshape(m, _C3_K)
    mo = mv - 4
    for kh in range(5):
        lhs = x2[kh:kh + mo]
        for sl, acc in ((16, acc_e), (0, acc_o)):
            rhs = t_ref[kh, sl:sl + _C3_K, :]
            z = jnp.dot(lhs, rhs, preferred_element_type=jnp.float32)
            if kh == 0:
                acc[...] = z
            else:
                acc[...] += z
    bias = b_ref[...]
    ae = jnp.maximum(acc_e[...] + bias, 0.0)
    ao = jnp.maximum(acc_o[...] + bias, 0.0)
    pw = jnp.maximum(ae, ao)
    for i in range(_C3_IMGS):
        o_ref[i] = pw[i * _C3_ROWS:i * _C3_ROWS + 56].reshape(28, 2, _C3_NL).max(axis=1)


def _conv3(xf, w_taps, b_row):
    n = xf.shape[0]
    t3 = _toep3(w_taps)
    bt = jnp.tile(jnp.pad(b_row, ((0, 0), (0, 8))), (1, 7))
    return pl.pallas_call(
        _c3_body,
        out_shape=jax.ShapeDtypeStruct((n, 28, 28 * 128), jnp.float32),
        grid=(_C3_NT, n // _C3_IMGS),
        in_specs=[
            pl.BlockSpec((_C3_IMGS, _C3_ROWS, _C3_K), lambda t, ib: (ib, 0, 0)),
            pl.BlockSpec((5, 976, _C3_NL), lambda t, ib: (0, 0, t)),
            pl.BlockSpec((1, _C3_NL), lambda t, ib: (0, 0)),
        ],
        out_specs=pl.BlockSpec((_C3_IMGS, 28, _C3_NL), lambda t, ib: (ib, 0, t)),
        scratch_shapes=[pltpu.VMEM((504, _C3_NL), jnp.float32),
                        pltpu.VMEM((504, _C3_NL), jnp.float32)],
        compiler_params=pltpu.CompilerParams(
            dimension_semantics=("parallel", "arbitrary"),
            vmem_limit_bytes=48 * 1024 * 1024),
    )(xf, t3, bt)


def _mlp_body(x_ref, w1_ref, b1_ref, w2_ref, b2_ref, o_ref, acc_ref):
    k = pl.program_id(0)

    @pl.when(k == 0)
    def _():
        acc_ref[...] = jnp.zeros_like(acc_ref)

    acc_ref[...] += jnp.dot(x_ref[...], w1_ref[...],
                            preferred_element_type=jnp.float32)

    @pl.when(k == pl.num_programs(0) - 1)
    def _():
        h = jnp.maximum(acc_ref[...] + b1_ref[...], 0.0)
        o_ref[...] = jnp.dot(h, w2_ref[...],
                             preferred_element_type=jnp.float32) + b2_ref[...]


def _mlp_head(feats, w1, b1, w2, b2, *, tk):
    n, kdim = feats.shape
    h1 = w1.shape[1]
    o = w2.shape[1]
    return pl.pallas_call(
        _mlp_body,
        out_shape=jax.ShapeDtypeStruct((n, o), jnp.float32),
        grid=(kdim // tk,),
        in_specs=[
            pl.BlockSpec((n, tk), lambda k: (0, k)),
            pl.BlockSpec((tk, h1), lambda k: (k, 0)),
            pl.BlockSpec((1, h1), lambda k: (0, 0)),
            pl.BlockSpec((h1, o), lambda k: (0, 0)),
            pl.BlockSpec((1, o), lambda k: (0, 0)),
        ],
        out_specs=pl.BlockSpec((n, o), lambda k: (0, 0)),
        scratch_shapes=[pltpu.VMEM((n, h1), jnp.float32)],
        compiler_params=pltpu.CompilerParams(
            dimension_semantics=("arbitrary",),
            vmem_limit_bytes=48 * 1024 * 1024),
    )(feats, w1, b1, w2, b2)


def _pad_flat(y, wc):
    return jnp.pad(y, ((0, 0), (2, 2), (2 * wc, 2 * wc)))


def kernel(x_nchw, c1w, c1b, c2w, c2b, c3w, c3b, f1w, f1b, f2w, f2b):
    n = x_nchw.shape[0]
    x2 = _conv1(x_nchw, c1w, c1b)
    x3 = _conv2(x2, c2w, c2b)
    y3 = _conv3(x3, c3w, c3b)
    feats = y3.reshape(n, 28, 28, 128)[:, :, :, :120].reshape(n, 94080)
    return _mlp_head(feats, f1w, f1b, f2w, f2b, tk=18816)
